# baseline (speedup 1.0000x reference)
import functools

import jax
import jax.numpy as jnp
from jax import lax
from jax.experimental import pallas as pl
from jax.experimental.pallas import tpu as pltpu
from jax.experimental.pallas import tpu_sc as plsc

_N = 20000
_NS = 16
_PER = 1280
_CH = _PER // 16
_K = 100
_WMAX = 4
_SCORE_THRESH = 0.05
_NMS_THRESH = 0.5
_BIG = 2 ** 30


def _row(lane, entries):
    r = jnp.zeros((16,), jnp.float32)
    for pos, v in entries:
        r = jnp.where(lane == pos, v, r)
    return r


def _top2_update(sm, lidx, nv1, ni1, nv2, ni2):
    c1 = sm > nv1
    dem_v = jnp.where(c1, nv1, sm)
    dem_i = jnp.where(c1, ni1, lidx)
    nv1 = jnp.where(c1, sm, nv1)
    ni1 = jnp.where(c1, lidx, ni1)
    c2 = dem_v > nv2
    nv2 = jnp.where(c2, dem_v, nv2)
    ni2 = jnp.where(c2, dem_i, ni2)
    return nv1, ni1, nv2, ni2


def _nms_sc(x1_h, y1_h, x2_h, y2_h, s_h, out_h,
            x1_v, y1_v, x2_v, y2_v, s_v, area_v, stats_v, allstats_v, out_v,
            shared):
    sid = lax.axis_index("s")
    cid = lax.axis_index("c")
    base = sid * _PER

    pltpu.sync_copy(x1_h.at[pl.ds(base, _PER)], x1_v)
    pltpu.sync_copy(y1_h.at[pl.ds(base, _PER)], y1_v)
    pltpu.sync_copy(x2_h.at[pl.ds(base, _PER)], x2_v)
    pltpu.sync_copy(y2_h.at[pl.ds(base, _PER)], y2_v)
    pltpu.sync_copy(s_h.at[pl.ds(base, _PER)], s_v)

    lane = lax.broadcasted_iota(jnp.int32, (16,), 0)
    neg = jnp.float32(-jnp.inf)
    negv = jnp.full((16,), neg)
    bigv = jnp.full((16,), _BIG, jnp.int32)
    zv = jnp.zeros((16,), jnp.float32)

    nv1, ni1, nv2, ni2 = negv, lane, negv, lane
    lidx = lane
    for c in range(_CH):
        sl = pl.ds(c * 16, 16)
        sr = s_v[sl]
        sm = jnp.where(sr > _SCORE_THRESH, sr, negv)
        s_v[sl] = sm
        area_v[sl] = (x2_v[sl] - x1_v[sl]) * (y2_v[sl] - y1_v[sl])
        if c == 0:
            nv1, ni1 = sm, lidx
        else:
            nv1, ni1, nv2, ni2 = _top2_update(sm, lidx, nv1, ni1, nv2, ni2)
        lidx = lidx + 16

    def gat(idx):
        return plsc.load_gather(allstats_v, [idx])

    def body(carry):
        k, it, nv1, ni1, nv2, ni2 = carry

        gi1v = ni1 + base
        gi2v = ni2 + base
        m1 = jnp.max(nv1)
        m1v = jnp.full((16,), m1)
        i1 = jnp.min(jnp.where(nv1 == m1v, gi1v, bigv))
        selw = gi1v == jnp.full((16,), i1)
        u = jnp.where(selw, nv2, nv1)
        uig = jnp.where(selw, gi2v, gi1v)
        m2 = jnp.max(u)
        i2 = jnp.min(jnp.where(u == jnp.full((16,), m2), uig, bigv))

        l1v = jnp.full((16,), i1 - base, jnp.int32)
        l2v = jnp.full((16,), i2 - base, jnp.int32)
        c1x1 = plsc.load_gather(x1_v, [l1v])
        c1y1 = plsc.load_gather(y1_v, [l1v])
        c1x2 = plsc.load_gather(x2_v, [l1v])
        c1y2 = plsc.load_gather(y2_v, [l1v])
        c2x1 = plsc.load_gather(x1_v, [l2v])
        c2y1 = plsc.load_gather(y1_v, [l2v])
        c2x2 = plsc.load_gather(x2_v, [l2v])
        c2y2 = plsc.load_gather(y2_v, [l2v])
        i1f = jnp.full((16,), i1, jnp.int32).astype(jnp.float32)
        i2f = jnp.full((16,), i2, jnp.int32).astype(jnp.float32)
        stats_v[...] = _row(lane, [
            (0, m1v), (1, i1f), (2, c1x1), (3, c1y1), (4, c1x2), (5, c1y2),
            (6, jnp.full((16,), m2)), (7, i2f),
            (8, c2x1), (9, c2y1), (10, c2x2), (11, c2y2)])

        off = (it % 2) * 256
        pltpu.sync_copy(stats_v, shared.at[pl.ds(off + sid * 16, 16)])
        plsc.subcore_barrier()
        pltpu.sync_copy(shared.at[pl.ds(off, 256)], allstats_v)

        r16 = lane * 16
        va = gat(r16)
        iag = gat(r16 + 1).astype(jnp.int32)
        ax1, ay1 = gat(r16 + 2), gat(r16 + 3)
        ax2, ay2 = gat(r16 + 4), gat(r16 + 5)
        vb = gat(r16 + 6)
        ibg = gat(r16 + 7).astype(jnp.int32)
        bx1, by1 = gat(r16 + 8), gat(r16 + 9)
        bx2, by2 = gat(r16 + 10), gat(r16 + 11)
        aareav = (ax2 - ax1) * (ay2 - ay1)
        bareav = (bx2 - bx1) * (by2 - by1)

        sa, sb = va, vb
        ub = neg
        live = None
        winners = []
        counts = []
        for j in range(_WMAX):
            m = jnp.maximum(jnp.max(sa), jnp.max(sb))
            mspl = jnp.full((16,), m)
            ca = jnp.min(jnp.where(sa == mspl, iag, bigv))
            cb = jnp.min(jnp.where(sb == mspl, ibg, bigv))
            gi = jnp.minimum(ca, cb)
            has = m > neg
            if j == 0:
                valid = None
            else:
                valid = live & (m > ub) & has & (k + j < _K)
                live = valid

            t16 = (gi // _PER) * 16
            t16v = jnp.full((16,), t16, jnp.int32)
            i1t = gat(t16v + 1).astype(jnp.int32)
            isa = i1t == jnp.full((16,), gi)
            colb = jnp.where(isa, jnp.full((16,), 2, jnp.int32),
                             jnp.full((16,), 8, jnp.int32))
            wx1 = gat(t16v + colb)
            wy1 = gat(t16v + colb + 1)
            wx2 = gat(t16v + colb + 2)
            wy2 = gat(t16v + colb + 3)

            if j == 0:
                condv = mspl > negv
                live = has
            else:
                vf = jnp.where(valid, jnp.float32(1.0), jnp.float32(0.0))
                condv = jnp.full((16,), vf) > 0.5
            sx1 = jnp.where(condv, wx1, jnp.full((16,), 5000.0))
            sy1 = jnp.where(condv, wy1, jnp.full((16,), 5000.0))
            sx2 = jnp.where(condv, wx2, jnp.full((16,), 4999.0))
            sy2 = jnp.where(condv, wy2, jnp.full((16,), 4999.0))
            wareav = (sx2 - sx1) * (sy2 - sy1)
            winners.append((sx1, sy1, sx2, sy2, wareav))

            orow = jnp.where(condv, _row(lane, [
                (0, sx1), (1, sy1), (2, sx2), (3, sy2), (4, mspl)]), zv)
            if j == 0:
                out_v[pl.ds(k * 16, 16)] = orow
            else:
                counts.append(valid)

                @pl.when(valid)
                def _(orow=orow, j=j):
                    out_v[pl.ds((k + j) * 16, 16)] = orow

            ixa1 = jnp.maximum(sx1, ax1)
            iya1 = jnp.maximum(sy1, ay1)
            ixa2 = jnp.minimum(sx2, ax2)
            iya2 = jnp.minimum(sy2, ay2)
            inta = (jnp.maximum(ixa2 - ixa1, 0.0) *
                    jnp.maximum(iya2 - iya1, 0.0))
            ioua = inta / jnp.maximum(wareav + aareav - inta, 1e-8)
            sa = jnp.where(ioua > _NMS_THRESH, negv, sa)
            ixb1 = jnp.maximum(sx1, bx1)
            iyb1 = jnp.maximum(sy1, by1)
            ixb2 = jnp.minimum(sx2, bx2)
            iyb2 = jnp.minimum(sy2, by2)
            intb = (jnp.maximum(ixb2 - ixb1, 0.0) *
                    jnp.maximum(iyb2 - iyb1, 0.0))
            ioub = intb / jnp.maximum(wareav + bareav - intb, 1e-8)
            sb = jnp.where(ioub > _NMS_THRESH, negv, sb)
            ub = jnp.maximum(ub, jnp.max(jnp.where(sb == negv, vb, negv)))

        count = jnp.int32(1)
        for v in counts:
            count = count + v.astype(jnp.int32)

        nv1n, ni1n, nv2n, ni2n = negv, lane, negv, lane
        lidx = lane
        for c in range(_CH):
            sl = pl.ds(c * 16, 16)
            cx1 = x1_v[sl]
            cy1 = y1_v[sl]
            cx2 = x2_v[sl]
            cy2 = y2_v[sl]
            car = area_v[sl]
            cs = s_v[sl]
            supp = None
            for (sx1, sy1, sx2, sy2, wareav) in winners:
                ix1 = jnp.maximum(sx1, cx1)
                iy1 = jnp.maximum(sy1, cy1)
                ix2 = jnp.minimum(sx2, cx2)
                iy2 = jnp.minimum(sy2, cy2)
                inter = (jnp.maximum(ix2 - ix1, 0.0) *
                         jnp.maximum(iy2 - iy1, 0.0))
                iou = inter / jnp.maximum(wareav + car - inter, 1e-8)
                s_j = iou > _NMS_THRESH
                supp = s_j if supp is None else (supp | s_j)
            snew = jnp.where(supp, negv, cs)
            s_v[sl] = snew
            if c == 0:
                nv1n, ni1n = snew, lidx
            else:
                nv1n, ni1n, nv2n, ni2n = _top2_update(
                    snew, lidx, nv1n, ni1n, nv2n, ni2n)
            lidx = lidx + 16
        return (k + count, it + 1, nv1n, ni1n, nv2n, ni2n)

    lax.while_loop(lambda c: c[0] < _K, body,
                   (jnp.int32(0), jnp.int32(0), nv1, ni1, nv2, ni2))

    @pl.when((sid == 0) & (cid == 0))
    def _():
        pltpu.sync_copy(out_v, out_h)


def kernel(boxes, scores):
    pad = _NS * _PER - _N
    bt = jnp.pad(jnp.transpose(boxes), ((0, 0), (0, pad)))
    s = jnp.pad(scores, (0, pad))

    mesh = plsc.VectorSubcoreMesh(
        core_axis_name="c", subcore_axis_name="s", num_cores=2)
    f = functools.partial(
        pl.kernel,
        mesh=mesh,
        compiler_params=pltpu.CompilerParams(needs_layout_passes=False),
        out_type=jax.ShapeDtypeStruct((_K * 16,), jnp.float32),
        scratch_types=[
            pltpu.VMEM((_PER,), jnp.float32),
            pltpu.VMEM((_PER,), jnp.float32),
            pltpu.VMEM((_PER,), jnp.float32),
            pltpu.VMEM((_PER,), jnp.float32),
            pltpu.VMEM((_PER,), jnp.float32),
            pltpu.VMEM((_PER,), jnp.float32),
            pltpu.VMEM((16,), jnp.float32),
            pltpu.VMEM((256,), jnp.float32),
            pltpu.VMEM((_K * 16,), jnp.float32),
            pltpu.VMEM_SHARED((512,), jnp.float32),
        ],
    )(_nms_sc)
    out = f(bt[0], bt[1], bt[2], bt[3], s)
    return out.reshape(_K, 16)[:, :5]


# speedup vs baseline: 1.0276x; 1.0276x over previous
import functools

import jax
import jax.numpy as jnp
from jax import lax
from jax.experimental import pallas as pl
from jax.experimental.pallas import tpu as pltpu
from jax.experimental.pallas import tpu_sc as plsc

_N = 20000
_NS = 16
_PER = 1280
_CH = _PER // 16
_K = 100
_SCORE_THRESH = 0.05
_NMS_THRESH = 0.5
_BIG = 2 ** 30


def _row6(lane, a, b, c, d, e, f):
    z = jnp.zeros((16,), jnp.float32)
    r = jnp.where(lane == 0, a, z)
    r = jnp.where(lane == 1, b, r)
    r = jnp.where(lane == 2, c, r)
    r = jnp.where(lane == 3, d, r)
    r = jnp.where(lane == 4, e, r)
    r = jnp.where(lane == 5, f, r)
    return r


def _nms_sc(x1_h, y1_h, x2_h, y2_h, s_h, out_h,
            x1_v, y1_v, x2_v, y2_v, s_v, area_v, stats_v, allstats_v, out_v,
            shared):
    sid = lax.axis_index("s")
    cid = lax.axis_index("c")
    base = sid * _PER

    pltpu.sync_copy(x1_h.at[pl.ds(base, _PER)], x1_v)
    pltpu.sync_copy(y1_h.at[pl.ds(base, _PER)], y1_v)
    pltpu.sync_copy(x2_h.at[pl.ds(base, _PER)], x2_v)
    pltpu.sync_copy(y2_h.at[pl.ds(base, _PER)], y2_v)
    pltpu.sync_copy(s_h.at[pl.ds(base, _PER)], s_v)

    lane = lax.broadcasted_iota(jnp.int32, (16,), 0)
    neg = jnp.float32(-jnp.inf)
    negv = jnp.full((16,), neg)

    vmax = negv
    vidx = lane
    for c in range(_CH):
        sl = pl.ds(c * 16, 16)
        sr = s_v[sl]
        gidx = lane + (base + c * 16)
        ok = (sr > _SCORE_THRESH) & (gidx < _N)
        sm = jnp.where(ok, sr, negv)
        s_v[sl] = sm
        area_v[sl] = (x2_v[sl] - x1_v[sl]) * (y2_v[sl] - y1_v[sl])
        lidx = lane + c * 16
        if c == 0:
            vmax, vidx = sm, lidx
        else:
            cond = sm > vmax
            vmax = jnp.where(cond, sm, vmax)
            vidx = jnp.where(cond, lidx, vidx)

    def body(i, carry):
        vmax, vidx = carry
        m_l = jnp.max(vmax)
        i_l = jnp.min(jnp.where(vmax == m_l, vidx, jnp.full((16,), _BIG)))
        ginds = jnp.full((16,), i_l, jnp.int32)
        gx1 = plsc.load_gather(x1_v, [ginds])
        gy1 = plsc.load_gather(y1_v, [ginds])
        gx2 = plsc.load_gather(x2_v, [ginds])
        gy2 = plsc.load_gather(y2_v, [ginds])
        gidxf = jnp.full((16,), i_l + base, jnp.int32).astype(jnp.float32)
        stats_v[...] = _row6(lane, jnp.full((16,), m_l), gidxf, gx1, gy1, gx2, gy2)
        off = (i % 2) * 256
        pltpu.sync_copy(stats_v, shared.at[pl.ds(off + sid * 16, 16)])
        plsc.subcore_barrier()
        pltpu.sync_copy(shared.at[pl.ds(off, 256)], allstats_v)

        rows16 = lane * 16
        maxv = plsc.load_gather(allstats_v, [rows16])
        idxv = plsc.load_gather(allstats_v, [rows16 + 1]).astype(jnp.int32)
        m = jnp.max(maxv)
        gi = jnp.min(jnp.where(maxv == m, idxv, jnp.full((16,), _BIG)))
        w16 = (gi // _PER) * 16
        bx1v = plsc.load_gather(allstats_v, [jnp.full((16,), w16 + 2, jnp.int32)])
        by1v = plsc.load_gather(allstats_v, [jnp.full((16,), w16 + 3, jnp.int32)])
        bx2v = plsc.load_gather(allstats_v, [jnp.full((16,), w16 + 4, jnp.int32)])
        by2v = plsc.load_gather(allstats_v, [jnp.full((16,), w16 + 5, jnp.int32)])
        mv = jnp.full((16,), m)
        hasv = mv > negv
        z = jnp.zeros((16,), jnp.float32)

        orow = jnp.where(hasv, _row6(lane, bx1v, by1v, bx2v, by2v, mv, z), z)
        out_v[pl.ds(i * 16, 16)] = orow

        sx1 = jnp.where(hasv, bx1v, jnp.full((16,), 5000.0))
        sy1 = jnp.where(hasv, by1v, jnp.full((16,), 5000.0))
        sx2 = jnp.where(hasv, bx2v, jnp.full((16,), 4999.0))
        sy2 = jnp.where(hasv, by2v, jnp.full((16,), 4999.0))
        bareav = (sx2 - sx1) * (sy2 - sy1)

        nvmax = negv
        nvidx = lane
        for c in range(_CH):
            sl = pl.ds(c * 16, 16)
            cx1 = x1_v[sl]
            cy1 = y1_v[sl]
            cx2 = x2_v[sl]
            cy2 = y2_v[sl]
            cs = s_v[sl]
            ix1 = jnp.maximum(sx1, cx1)
            iy1 = jnp.maximum(sy1, cy1)
            ix2 = jnp.minimum(sx2, cx2)
            iy2 = jnp.minimum(sy2, cy2)
            inter = jnp.maximum(ix2 - ix1, 0.0) * jnp.maximum(iy2 - iy1, 0.0)
            union = bareav + area_v[sl] - inter
            iou = inter / jnp.maximum(union, 1e-8)
            snew = jnp.where(iou > _NMS_THRESH, negv, cs)
            s_v[sl] = snew
            lidx = lane + c * 16
            if c == 0:
                nvmax, nvidx = snew, lidx
            else:
                cond = snew > nvmax
                nvmax = jnp.where(cond, snew, nvmax)
                nvidx = jnp.where(cond, lidx, nvidx)
        return (nvmax, nvidx)

    lax.fori_loop(0, _K, body, (vmax, vidx))

    @pl.when((sid == 0) & (cid == 0))
    def _():
        pltpu.sync_copy(out_v, out_h)


def kernel(boxes, scores):
    pad = _NS * _PER - _N
    bt = jnp.pad(jnp.transpose(boxes), ((0, 0), (0, pad)))
    s = jnp.pad(scores, (0, pad))

    mesh = plsc.VectorSubcoreMesh(
        core_axis_name="c", subcore_axis_name="s", num_cores=2)
    f = functools.partial(
        pl.kernel,
        mesh=mesh,
        compiler_params=pltpu.CompilerParams(needs_layout_passes=False),
        out_type=jax.ShapeDtypeStruct((_K * 16,), jnp.float32),
        scratch_types=[
            pltpu.VMEM((_PER,), jnp.float32),
            pltpu.VMEM((_PER,), jnp.float32),
            pltpu.VMEM((_PER,), jnp.float32),
            pltpu.VMEM((_PER,), jnp.float32),
            pltpu.VMEM((_PER,), jnp.float32),
            pltpu.VMEM((_PER,), jnp.float32),
            pltpu.VMEM((16,), jnp.float32),
            pltpu.VMEM((256,), jnp.float32),
            pltpu.VMEM((_K * 16,), jnp.float32),
            pltpu.VMEM_SHARED((512,), jnp.float32),
        ],
    )(_nms_sc)
    out = f(bt[0], bt[1], bt[2], bt[3], s)
    return out.reshape(_K, 16)[:, :5]
